# baseline (speedup 1.0000x reference)
import jax
import jax.numpy as jnp
from jax import lax
from jax.experimental import pallas as pl
from jax.experimental.pallas import tpu as pltpu
from jax.experimental.pallas import tpu_sc as plsc

L = 20
BATCH = 16384

_NC = 2
_NS = 16
_NW = _NC * _NS
_ROWS = BATCH // _NW
_CHUNK = 128
_NCHUNK = _ROWS // _CHUNK
_GROUPS = _CHUNK // 16


def _ewf_body(xt_hbm, aux_hbm, out_hbm, x_v, idx_v, rows_v, xsem, xsem2, gsem):
    wid = lax.axis_index("s") * _NC + lax.axis_index("c")
    base = wid * _ROWS

    _HALF = _ROWS // 2
    xcp0 = pltpu.async_copy(
        xt_hbm.at[:, pl.ds(base, _HALF)], x_v.at[:, pl.ds(0, _HALF)], xsem
    )
    xcp1 = pltpu.async_copy(
        xt_hbm.at[:, pl.ds(base + _HALF, _HALF)],
        x_v.at[:, pl.ds(_HALF, _HALF)],
        xsem2,
    )

    half = jnp.full((16,), (2.0 ** L - 1.0) / 2.0, dtype=jnp.float32)
    xcp0.wait()

    def chunk(c, carry):
        @pl.when(c == _NCHUNK // 2)
        def _():
            xcp1.wait()

        @plsc.parallel_loop(0, _GROUPS, unroll=2)
        def group(g):
            off = c * _CHUNK + g * 16
            accs = [jnp.zeros((16,), jnp.float32) for _ in range(4)]
            for i in range(L):
                v = x_v[i, pl.ds(off, 16)]
                accs[i % 4] = accs[i % 4] + v * (2.0 ** (L - 2 - i))
            acc = (accs[0] + accs[1]) + (accs[2] + accs[3]) + half
            idx_v[pl.ds(off, 16)] = acc.astype(jnp.int32)

        pltpu.async_copy(
            aux_hbm.at[idx_v.at[pl.ds(c * _CHUNK, _CHUNK)]],
            rows_v.at[pl.ds(c * _CHUNK, _CHUNK)],
            gsem,
        )
        return carry

    lax.fori_loop(0, _NCHUNK, chunk, 0)

    pltpu.make_async_copy(aux_hbm.at[idx_v], rows_v, gsem).wait()

    pltpu.sync_copy(rows_v, out_hbm.at[pl.ds(base, _ROWS)])


@jax.jit
def _ewf(x, aux):
    mesh = plsc.VectorSubcoreMesh(
        core_axis_name="c", subcore_axis_name="s",
        num_cores=_NC, num_subcores=_NS,
    )
    return pl.kernel(
        _ewf_body,
        out_type=jax.ShapeDtypeStruct((BATCH,), jnp.float32),
        mesh=mesh,
        scratch_types=[
            pltpu.VMEM((L, _ROWS), jnp.float32),
            pltpu.VMEM((_ROWS,), jnp.int32),
            pltpu.VMEM((_ROWS,), jnp.float32),
            pltpu.SemaphoreType.DMA,
            pltpu.SemaphoreType.DMA,
            pltpu.SemaphoreType.DMA,
        ],
        compiler_params=pltpu.CompilerParams(
            needs_layout_passes=False, skip_device_barrier=True
        ),
    )(x.T, aux)


def kernel(x, aux, j1):
    del j1
    return _ewf(x, aux)


# speedup vs baseline: 1.0013x; 1.0013x over previous
import jax
import jax.numpy as jnp
from jax import lax
from jax.experimental import pallas as pl
from jax.experimental.pallas import tpu as pltpu
from jax.experimental.pallas import tpu_sc as plsc

L = 20
BATCH = 16384

_NC = 2
_NS = 16
_NW = _NC * _NS
_ROWS = BATCH // _NW
_CHUNK = 128
_NCHUNK = _ROWS // _CHUNK
_GROUPS = _CHUNK // 16


def _ewf_body(xt_hbm, aux_hbm, out_hbm, x_v, idx_v, rows_v, xsem, xsem2, gsem):
    wid = lax.axis_index("s") * _NC + lax.axis_index("c")
    base = wid * _ROWS

    _HALF = _ROWS // 2
    xcp0 = pltpu.async_copy(
        xt_hbm.at[:, pl.ds(base, _HALF)], x_v.at[:, pl.ds(0, _HALF)], xsem
    )
    xcp1 = pltpu.async_copy(
        xt_hbm.at[:, pl.ds(base + _HALF, _HALF)],
        x_v.at[:, pl.ds(_HALF, _HALF)],
        xsem2,
    )

    half = jnp.full((16,), (2.0 ** L - 1.0) / 2.0, dtype=jnp.float32)
    xcp0.wait()

    def chunk(c, carry):
        @pl.when(c == _NCHUNK // 2)
        def _():
            xcp1.wait()

        @plsc.parallel_loop(0, _GROUPS, unroll=1)
        def group(g):
            off = c * _CHUNK + g * 16
            accs = [jnp.zeros((16,), jnp.float32) for _ in range(4)]
            for i in range(L):
                v = x_v[i, pl.ds(off, 16)]
                accs[i % 4] = accs[i % 4] + v * (2.0 ** (L - 2 - i))
            acc = (accs[0] + accs[1]) + (accs[2] + accs[3]) + half
            idx_v[pl.ds(off, 16)] = acc.astype(jnp.int32)

        pltpu.async_copy(
            aux_hbm.at[idx_v.at[pl.ds(c * _CHUNK, _CHUNK)]],
            rows_v.at[pl.ds(c * _CHUNK, _CHUNK)],
            gsem,
        )
        return carry

    lax.fori_loop(0, _NCHUNK, chunk, 0)

    pltpu.make_async_copy(aux_hbm.at[idx_v], rows_v, gsem).wait()

    pltpu.sync_copy(rows_v, out_hbm.at[pl.ds(base, _ROWS)])


@jax.jit
def _ewf(x, aux):
    mesh = plsc.VectorSubcoreMesh(
        core_axis_name="c", subcore_axis_name="s",
        num_cores=_NC, num_subcores=_NS,
    )
    return pl.kernel(
        _ewf_body,
        out_type=jax.ShapeDtypeStruct((BATCH,), jnp.float32),
        mesh=mesh,
        scratch_types=[
            pltpu.VMEM((L, _ROWS), jnp.float32),
            pltpu.VMEM((_ROWS,), jnp.int32),
            pltpu.VMEM((_ROWS,), jnp.float32),
            pltpu.SemaphoreType.DMA,
            pltpu.SemaphoreType.DMA,
            pltpu.SemaphoreType.DMA,
        ],
        compiler_params=pltpu.CompilerParams(
            needs_layout_passes=False, skip_device_barrier=True
        ),
    )(x.T, aux)


def kernel(x, aux, j1):
    del j1
    return _ewf(x, aux)


# speedup vs baseline: 1.0036x; 1.0024x over previous
import jax
import jax.numpy as jnp
from jax import lax
from jax.experimental import pallas as pl
from jax.experimental.pallas import tpu as pltpu
from jax.experimental.pallas import tpu_sc as plsc

L = 20
BATCH = 16384

_NC = 2
_NS = 16
_NW = _NC * _NS
_ROWS = BATCH // _NW
_CHUNK = 128
_NCHUNK = _ROWS // _CHUNK
_GROUPS = _CHUNK // 16


def _ewf_body(xt_hbm, aux_hbm, out_hbm, x_v, idx_v, rows_v, xsem, xsem2, gsem):
    wid = lax.axis_index("s") * _NC + lax.axis_index("c")
    base = wid * _ROWS

    xcp0 = pltpu.async_copy(
        xt_hbm.at[:, pl.ds(base, _CHUNK)], x_v.at[:, pl.ds(0, _CHUNK)], xsem
    )
    xcp1 = pltpu.async_copy(
        xt_hbm.at[:, pl.ds(base + _CHUNK, _ROWS - _CHUNK)],
        x_v.at[:, pl.ds(_CHUNK, _ROWS - _CHUNK)],
        xsem2,
    )

    half = jnp.full((16,), (2.0 ** L - 1.0) / 2.0, dtype=jnp.float32)
    xcp0.wait()

    def chunk(c, carry):
        @pl.when(c == 1)
        def _():
            xcp1.wait()

        @plsc.parallel_loop(0, _GROUPS, unroll=1)
        def group(g):
            off = c * _CHUNK + g * 16
            accs = [jnp.zeros((16,), jnp.float32) for _ in range(4)]
            for i in range(L):
                v = x_v[i, pl.ds(off, 16)]
                accs[i % 4] = accs[i % 4] + v * (2.0 ** (L - 2 - i))
            acc = (accs[0] + accs[1]) + (accs[2] + accs[3]) + half
            idx_v[pl.ds(off, 16)] = acc.astype(jnp.int32)

        pltpu.async_copy(
            aux_hbm.at[idx_v.at[pl.ds(c * _CHUNK, _CHUNK)]],
            rows_v.at[pl.ds(c * _CHUNK, _CHUNK)],
            gsem,
        )
        return carry

    lax.fori_loop(0, _NCHUNK, chunk, 0)

    pltpu.make_async_copy(aux_hbm.at[idx_v], rows_v, gsem).wait()

    pltpu.sync_copy(rows_v, out_hbm.at[pl.ds(base, _ROWS)])


@jax.jit
def _ewf(x, aux):
    mesh = plsc.VectorSubcoreMesh(
        core_axis_name="c", subcore_axis_name="s",
        num_cores=_NC, num_subcores=_NS,
    )
    return pl.kernel(
        _ewf_body,
        out_type=jax.ShapeDtypeStruct((BATCH,), jnp.float32),
        mesh=mesh,
        scratch_types=[
            pltpu.VMEM((L, _ROWS), jnp.float32),
            pltpu.VMEM((_ROWS,), jnp.int32),
            pltpu.VMEM((_ROWS,), jnp.float32),
            pltpu.SemaphoreType.DMA,
            pltpu.SemaphoreType.DMA,
            pltpu.SemaphoreType.DMA,
        ],
        compiler_params=pltpu.CompilerParams(
            needs_layout_passes=False, skip_device_barrier=True
        ),
    )(x.T, aux)


def kernel(x, aux, j1):
    del j1
    return _ewf(x, aux)
